# Initial kernel scaffold; baseline (speedup 1.0000x reference)
import jax
import jax.numpy as jnp
from jax.experimental import pallas as pl


def kernel(x, edge_index, batch, W1l, b1l, W1r, W2l, b2l, W2r, Wa1, ba1, Wa2, ba2, We1, be1, We2, be2):
    raise NotImplementedError("write your pallas kernel here")


# speedup vs baseline: 4.1744x; 4.1744x over previous
import functools

import jax
import jax.numpy as jnp
from jax import lax
from jax.experimental import pallas as pl
from jax.experimental.pallas import tpu as pltpu
from jax.experimental.pallas import tpu_sc as plsc

N = 50000
E = 800000
IN = 11
H = 64
B = 64

D16 = 16
NACC = 50176
STRIPE = NACC // 16
EP = 819200
EROWS = EP // 128
G = 8
CHUNK = G * 128

_mesh = plsc.VectorSubcoreMesh(core_axis_name="c", subcore_axis_name="s")
_sc_params = pltpu.CompilerParams(use_tc_tiling_on_sc=False)


def _seg_chunk_loop(table, src2d, dst2d, acc, sidx, didx, rows, sem,
                    row_base, niter):
    def body(i, _):
        rb = row_base + i * G
        pltpu.sync_copy(src2d.at[pl.ds(rb, G)], sidx)
        pltpu.sync_copy(dst2d.at[pl.ds(rb, G)], didx)
        cps = [pltpu.async_copy(table.at[sidx.at[j]],
                                rows.at[pl.ds(j * 128, 128)], sem)
               for j in range(G)]
        for cp in cps:
            cp.wait()
        for j in range(G):
            pltpu.sync_copy(rows.at[pl.ds(j * 128, 128)],
                            acc.at[didx.at[j]], add=True)
        return 0
    lax.fori_loop(0, niter, body, 0)


def _make_seg_kernel(split_edges):
    rows_sc = EROWS // 2 if split_edges else EROWS
    rows_tile = rows_sc // 16
    niter = rows_tile // G

    @functools.partial(
        pl.kernel, mesh=_mesh, compiler_params=_sc_params,
        out_type=jax.ShapeDtypeStruct((2 * NACC, D16), jnp.float32),
        scratch_types=[
            pltpu.VMEM((G, 128), jnp.int32),
            pltpu.VMEM((G, 128), jnp.int32),
            pltpu.VMEM((CHUNK, D16), jnp.float32),
            pltpu.VMEM_SHARED((NACC, D16), jnp.float32),
            pltpu.SemaphoreType.DMA,
        ],
    )
    def k(t0, t1, src2d, dst2d, zeros_hbm, out, sidx, didx, rows, acc, sem):
        c = lax.axis_index("c")
        s = lax.axis_index("s")
        sb = s * STRIPE
        pltpu.sync_copy(zeros_hbm.at[pl.ds(sb, STRIPE)],
                        acc.at[pl.ds(sb, STRIPE)])
        plsc.subcore_barrier()
        base = s * rows_tile
        if split_edges:
            base = base + c * rows_sc

        @pl.when(c == 0)
        def _():
            _seg_chunk_loop(t0, src2d, dst2d, acc, sidx, didx, rows, sem,
                            base, niter)

        @pl.when(c == 1)
        def _():
            _seg_chunk_loop(t1, src2d, dst2d, acc, sidx, didx, rows, sem,
                            base, niter)

        plsc.subcore_barrier()
        pltpu.sync_copy(acc.at[pl.ds(sb, STRIPE)],
                        out.at[pl.ds(c * NACC + sb, STRIPE)])

    return k


_seg_split = _make_seg_kernel(True)
_seg_full = _make_seg_kernel(False)

_EROWS_TILE = EROWS // 32
_ENITER = _EROWS_TILE // G


@functools.partial(
    pl.kernel, mesh=_mesh, compiler_params=_sc_params,
    out_type=jax.ShapeDtypeStruct((2 * EP, 32), jnp.float32),
    scratch_types=[
        pltpu.VMEM((G, 128), jnp.int32),
        pltpu.VMEM((G, 128), jnp.int32),
        pltpu.VMEM((CHUNK, 32), jnp.float32),
        pltpu.VMEM((CHUNK, 32), jnp.float32),
        pltpu.SemaphoreType.DMA,
    ],
)
def _edge_gather(p_tab, q_tab, src2d, dst2d, out, sidx, didx, bufp, bufq, sem):
    c = lax.axis_index("c")
    s = lax.axis_index("s")
    w = s * 2 + c

    def body(i, _):
        rb = w * _EROWS_TILE + i * G
        eb = rb * 128
        pltpu.sync_copy(src2d.at[pl.ds(rb, G)], sidx)
        pltpu.sync_copy(dst2d.at[pl.ds(rb, G)], didx)
        cps = [pltpu.async_copy(p_tab.at[sidx.at[j]],
                                bufp.at[pl.ds(j * 128, 128)], sem)
               for j in range(G)]
        cps += [pltpu.async_copy(q_tab.at[didx.at[j]],
                                 bufq.at[pl.ds(j * 128, 128)], sem)
                for j in range(G)]
        for cp in cps:
            cp.wait()
        pltpu.sync_copy(bufp, out.at[pl.ds(eb, CHUNK)])
        pltpu.sync_copy(bufq, out.at[pl.ds(EP + eb, CHUNK)])
        return 0
    lax.fori_loop(0, _ENITER, body, 0)


BLK = 2000
_NB = N // BLK


def _tc1_body(a_ref, x_ref, wl_ref, wr_ref, bl_ref,
              q0_ref, q1_ref, q2_ref, q3_ref, ci_ref):
    p = a_ref[0] + a_ref[1]
    cnt = p[:, D16 - 1:D16]
    ci = 1.0 / jnp.maximum(cnt, 1.0)
    mean = p[:, :IN] * ci
    h = jax.nn.relu(mean @ wl_ref[...] + x_ref[...] @ wr_ref[...] + bl_ref[...])
    q0_ref[...] = h[:, 0:16]
    q1_ref[...] = h[:, 16:32]
    q2_ref[...] = h[:, 32:48]
    q3_ref[...] = h[:, 48:64]
    ci_ref[...] = ci


def _tc1(agg1, x, w1lt, w1rt, b1l):
    qshape = jax.ShapeDtypeStruct((N, 16), jnp.float32)
    qspec = pl.BlockSpec((BLK, 16), lambda i: (i, 0))
    return pl.pallas_call(
        _tc1_body,
        grid=(_NB,),
        in_specs=[
            pl.BlockSpec((2, BLK, D16), lambda i: (0, i, 0)),
            pl.BlockSpec((BLK, IN), lambda i: (i, 0)),
            pl.BlockSpec((IN, H), lambda i: (0, 0)),
            pl.BlockSpec((IN, H), lambda i: (0, 0)),
            pl.BlockSpec((1, H), lambda i: (0, 0)),
        ],
        out_specs=[qspec, qspec, qspec, qspec,
                   pl.BlockSpec((BLK, 1), lambda i: (i, 0))],
        out_shape=[qshape, qshape, qshape, qshape,
                   jax.ShapeDtypeStruct((N, 1), jnp.float32)],
    )(agg1, x, w1lt, w1rt, b1l)


def _tc2_body(a01_ref, a23_ref, q0_ref, q1_ref, q2_ref, q3_ref, ci_ref,
              bt_ref, w2lt, w2rt, b2l,
              wst, wdt, wa1t, ba1, wa2t, ba2,
              p_ref, q_ref, gsum_ref, gcnt_ref, act_ref):
    i = pl.program_id(0)
    ci = ci_ref[...]
    acc = b2l[...]
    aggs = [a01_ref[0], a01_ref[1], a23_ref[0], a23_ref[1]]
    roots = [q0_ref[...], q1_ref[...], q2_ref[...], q3_ref[...]]
    for qi in range(4):
        acc = acc + (aggs[qi] * ci) @ w2lt[qi]
        acc = acc + roots[qi] @ w2rt[qi]
    h2 = jax.nn.relu(acc)
    p_ref[...] = h2 @ wst[...]
    q_ref[...] = h2 @ wdt[...]
    bt = bt_ref[0, 0, :]
    ot = (lax.broadcasted_iota(jnp.int32, (B, BLK), 0)
          == bt[None, :]).astype(jnp.float32)

    @pl.when(i == 0)
    def _():
        gsum_ref[...] = jnp.zeros_like(gsum_ref)
        gcnt_ref[...] = jnp.zeros_like(gcnt_ref)

    gsum_ref[...] += ot @ h2
    gcnt_ref[...] += jnp.sum(ot, axis=1, keepdims=True)

    @pl.when(i == _NB - 1)
    def _():
        gm = gsum_ref[...] / jnp.maximum(gcnt_ref[...], 1.0)
        act_ref[...] = (jax.nn.relu(gm @ wa1t[...] + ba1[...])
                        @ wa2t[...] + ba2[...])


def _tc2(agg01, agg23, q0, q1, q2, q3, cinv, batch3, w2lt, w2rt, b2l,
         wst, wdt, wa1t, ba1, wa2t, ba2):
    const = lambda i: (0, 0)
    const3 = lambda i: (0, 0, 0)
    qspec = pl.BlockSpec((BLK, 16), lambda i: (i, 0))
    return pl.pallas_call(
        _tc2_body,
        grid=(_NB,),
        in_specs=[
            pl.BlockSpec((2, BLK, 16), lambda i: (0, i, 0)),
            pl.BlockSpec((2, BLK, 16), lambda i: (0, i, 0)),
            qspec, qspec, qspec, qspec,
            pl.BlockSpec((BLK, 1), lambda i: (i, 0)),
            pl.BlockSpec((1, 1, BLK), lambda i: (i, 0, 0)),
            pl.BlockSpec((4, 16, H), const3),
            pl.BlockSpec((4, 16, H), const3),
            pl.BlockSpec((1, H), const),
            pl.BlockSpec((H, 32), const),
            pl.BlockSpec((H, 32), const),
            pl.BlockSpec((H, 32), const),
            pl.BlockSpec((1, 32), const),
            pl.BlockSpec((32, 3), const),
            pl.BlockSpec((1, 3), const),
        ],
        out_specs=[
            pl.BlockSpec((BLK, 32), lambda i: (i, 0)),
            pl.BlockSpec((BLK, 32), lambda i: (i, 0)),
            pl.BlockSpec((B, H), const),
            pl.BlockSpec((B, 1), const),
            pl.BlockSpec((B, 3), const),
        ],
        out_shape=[
            jax.ShapeDtypeStruct((N, 32), jnp.float32),
            jax.ShapeDtypeStruct((N, 32), jnp.float32),
            jax.ShapeDtypeStruct((B, H), jnp.float32),
            jax.ShapeDtypeStruct((B, 1), jnp.float32),
            jax.ShapeDtypeStruct((B, 3), jnp.float32),
        ],
    )(agg01, agg23, q0, q1, q2, q3, cinv, batch3, w2lt, w2rt, b2l,
      wst, wdt, wa1t, ba1, wa2t, ba2)


EBLK = 4096
_NEB = EP // EBLK


def _tc3_body(r_ref, be1_ref, we2t_ref, be2_ref, out_ref):
    r = jax.nn.relu(r_ref[0] + r_ref[1] + be1_ref[...])
    out_ref[...] = r @ we2t_ref[...] + be2_ref[...]


def _tc3(r2, be1, we2t, be2):
    return pl.pallas_call(
        _tc3_body,
        grid=(_NEB,),
        in_specs=[
            pl.BlockSpec((2, EBLK, 32), lambda i: (0, i, 0)),
            pl.BlockSpec((1, 32), lambda i: (0, 0)),
            pl.BlockSpec((32, 1), lambda i: (0, 0)),
            pl.BlockSpec((1, 1), lambda i: (0, 0)),
        ],
        out_specs=pl.BlockSpec((EBLK, 1), lambda i: (i, 0)),
        out_shape=jax.ShapeDtypeStruct((EP, 1), jnp.float32),
    )(r2, be1, we2t, be2)


def kernel(x, edge_index, batch, W1l, b1l, W1r, W2l, b2l, W2r,
           Wa1, ba1, Wa2, ba2, We1, be1, We2, be2):
    f32 = jnp.float32
    x_pad = jnp.zeros((N, D16), f32)
    x_pad = x_pad.at[:, :IN].set(x).at[:, D16 - 1].set(1.0)

    src = jnp.concatenate(
        [edge_index[0], jnp.zeros((EP - E,), jnp.int32)]).reshape(EROWS, 128)
    dst = jnp.concatenate(
        [edge_index[1], jnp.full((EP - E,), N, jnp.int32)]).reshape(EROWS, 128)
    zeros16 = jnp.zeros((NACC, D16), f32)

    agg1 = _seg_split(x_pad, x_pad, src, dst, zeros16).reshape(2, NACC, D16)
    q0, q1, q2, q3, cinv = _tc1(agg1, x, W1l.T, W1r.T, b1l[None, :])

    agg01 = _seg_full(q0, q1, src, dst, zeros16).reshape(2, NACC, 16)
    agg23 = _seg_full(q2, q3, src, dst, zeros16).reshape(2, NACC, 16)

    w2lt = jnp.stack([W2l[:, 0:16].T, W2l[:, 16:32].T,
                      W2l[:, 32:48].T, W2l[:, 48:64].T])
    w2rt = jnp.stack([W2r[:, 0:16].T, W2r[:, 16:32].T,
                      W2r[:, 32:48].T, W2r[:, 48:64].T])
    batch3 = batch.reshape(_NB, 1, BLK)
    p_tab, q_tab, _gs, _gc, act = _tc2(
        agg01, agg23, q0, q1, q2, q3, cinv, batch3, w2lt, w2rt,
        b2l[None, :],
        We1[:, :H].T, We1[:, H:].T, Wa1.T, ba1[None, :], Wa2.T, ba2[None, :])

    r2 = _edge_gather(p_tab, q_tab, src, dst).reshape(2, EP, 32)
    el = _tc3(r2, be1[None, :], We2.T, be2[None, :])
    return act, el[:E, 0]


# speedup vs baseline: 4.5536x; 1.0908x over previous
import functools

import jax
import jax.numpy as jnp
from jax import lax
from jax.experimental import pallas as pl
from jax.experimental.pallas import tpu as pltpu
from jax.experimental.pallas import tpu_sc as plsc

N = 50000
E = 800000
IN = 11
H = 64
B = 64

D16 = 16
NACC = 50176
STRIPE = NACC // 16
EP = 819200
EROWS = EP // 128
G = 8
CHUNK = G * 128

_mesh = plsc.VectorSubcoreMesh(core_axis_name="c", subcore_axis_name="s")
_sc_params = pltpu.CompilerParams(use_tc_tiling_on_sc=False)


def _seg_chunk_loop(table, src2d, dst2d, acc, slots, row_base, niter):
    def fire(slot, k, guard_drain):
        sidx, didx, rows, gsem, ssem = slot
        if guard_drain:
            @pl.when(k >= 2)
            def _():
                for j in range(G):
                    pltpu.make_async_copy(rows.at[pl.ds(j * 128, 128)],
                                          acc.at[didx.at[j]], ssem).wait()
        rb = row_base + k * G
        pltpu.sync_copy(src2d.at[pl.ds(rb, G)], sidx)
        pltpu.sync_copy(dst2d.at[pl.ds(rb, G)], didx)
        for j in range(G):
            pltpu.async_copy(table.at[sidx.at[j]],
                             rows.at[pl.ds(j * 128, 128)], gsem)

    def work(slot):
        sidx, didx, rows, gsem, ssem = slot
        for j in range(G):
            pltpu.make_async_copy(table.at[sidx.at[j]],
                                  rows.at[pl.ds(j * 128, 128)], gsem).wait()
        for j in range(G):
            pltpu.async_copy(rows.at[pl.ds(j * 128, 128)],
                             acc.at[didx.at[j]], ssem, add=True)

    fire(slots[0], row_base * 0, False)

    def body(i, _):
        even = i % 2 == 0

        @pl.when(jnp.logical_and(even, i + 1 < niter))
        def _():
            fire(slots[1], i + 1, True)

        @pl.when(jnp.logical_and(jnp.logical_not(even), i + 1 < niter))
        def _():
            fire(slots[0], i + 1, True)

        @pl.when(even)
        def _():
            work(slots[0])

        @pl.when(jnp.logical_not(even))
        def _():
            work(slots[1])

        return 0
    lax.fori_loop(0, niter, body, 0)
    for slot in slots:
        sidx, didx, rows, gsem, ssem = slot
        for j in range(G):
            pltpu.make_async_copy(rows.at[pl.ds(j * 128, 128)],
                                  acc.at[didx.at[j]], ssem).wait()


def _make_seg_kernel(split_edges):
    rows_sc = EROWS // 2 if split_edges else EROWS
    rows_tile = rows_sc // 16
    niter = rows_tile // G

    @functools.partial(
        pl.kernel, mesh=_mesh, compiler_params=_sc_params,
        out_type=jax.ShapeDtypeStruct((2 * NACC, D16), jnp.float32),
        scratch_types=[
            pltpu.VMEM((G, 128), jnp.int32),
            pltpu.VMEM((G, 128), jnp.int32),
            pltpu.VMEM((CHUNK, D16), jnp.float32),
            pltpu.VMEM((G, 128), jnp.int32),
            pltpu.VMEM((G, 128), jnp.int32),
            pltpu.VMEM((CHUNK, D16), jnp.float32),
            pltpu.VMEM_SHARED((NACC, D16), jnp.float32),
            pltpu.SemaphoreType.DMA,
            pltpu.SemaphoreType.DMA,
            pltpu.SemaphoreType.DMA,
            pltpu.SemaphoreType.DMA,
        ],
    )
    def k(t0, t1, src2d, dst2d, zeros_hbm, out,
          si0, di0, rw0, si1, di1, rw1, acc, g0, x0, g1, x1):
        c = lax.axis_index("c")
        s = lax.axis_index("s")
        sb = s * STRIPE
        pltpu.sync_copy(zeros_hbm.at[pl.ds(sb, STRIPE)],
                        acc.at[pl.ds(sb, STRIPE)])
        plsc.subcore_barrier()
        base = s * rows_tile
        if split_edges:
            base = base + c * rows_sc
        slots = [(si0, di0, rw0, g0, x0), (si1, di1, rw1, g1, x1)]

        @pl.when(c == 0)
        def _():
            _seg_chunk_loop(t0, src2d, dst2d, acc, slots, base, niter)

        @pl.when(c == 1)
        def _():
            _seg_chunk_loop(t1, src2d, dst2d, acc, slots, base, niter)

        plsc.subcore_barrier()
        pltpu.sync_copy(acc.at[pl.ds(sb, STRIPE)],
                        out.at[pl.ds(c * NACC + sb, STRIPE)])

    return k


_seg_split = _make_seg_kernel(True)
_seg_full = _make_seg_kernel(False)

_EROWS_TILE = EROWS // 32
_ENITER = _EROWS_TILE // G


@functools.partial(
    pl.kernel, mesh=_mesh, compiler_params=_sc_params,
    out_type=jax.ShapeDtypeStruct((2 * EP, 32), jnp.float32),
    scratch_types=[
        pltpu.VMEM((G, 128), jnp.int32),
        pltpu.VMEM((G, 128), jnp.int32),
        pltpu.VMEM((CHUNK, 32), jnp.float32),
        pltpu.VMEM((CHUNK, 32), jnp.float32),
        pltpu.SemaphoreType.DMA,
    ],
)
def _edge_gather(p_tab, q_tab, src2d, dst2d, out, sidx, didx, bufp, bufq, sem):
    c = lax.axis_index("c")
    s = lax.axis_index("s")
    w = s * 2 + c

    def body(i, _):
        rb = w * _EROWS_TILE + i * G
        eb = rb * 128
        pltpu.sync_copy(src2d.at[pl.ds(rb, G)], sidx)
        pltpu.sync_copy(dst2d.at[pl.ds(rb, G)], didx)
        cps = [pltpu.async_copy(p_tab.at[sidx.at[j]],
                                bufp.at[pl.ds(j * 128, 128)], sem)
               for j in range(G)]
        cps += [pltpu.async_copy(q_tab.at[didx.at[j]],
                                 bufq.at[pl.ds(j * 128, 128)], sem)
                for j in range(G)]
        for cp in cps:
            cp.wait()
        pltpu.sync_copy(bufp, out.at[pl.ds(eb, CHUNK)])
        pltpu.sync_copy(bufq, out.at[pl.ds(EP + eb, CHUNK)])
        return 0
    lax.fori_loop(0, _ENITER, body, 0)


BLK = 2000
_NB = N // BLK


def _tc1_body(a_ref, x_ref, wl_ref, wr_ref, bl_ref,
              q0_ref, q1_ref, q2_ref, q3_ref, ci_ref):
    p = a_ref[0] + a_ref[1]
    cnt = p[:, D16 - 1:D16]
    ci = 1.0 / jnp.maximum(cnt, 1.0)
    mean = p[:, :IN] * ci
    h = jax.nn.relu(mean @ wl_ref[...] + x_ref[...] @ wr_ref[...] + bl_ref[...])
    q0_ref[...] = h[:, 0:16]
    q1_ref[...] = h[:, 16:32]
    q2_ref[...] = h[:, 32:48]
    q3_ref[...] = h[:, 48:64]
    ci_ref[...] = ci


def _tc1(agg1, x, w1lt, w1rt, b1l):
    qshape = jax.ShapeDtypeStruct((N, 16), jnp.float32)
    qspec = pl.BlockSpec((BLK, 16), lambda i: (i, 0))
    return pl.pallas_call(
        _tc1_body,
        grid=(_NB,),
        in_specs=[
            pl.BlockSpec((2, BLK, D16), lambda i: (0, i, 0)),
            pl.BlockSpec((BLK, IN), lambda i: (i, 0)),
            pl.BlockSpec((IN, H), lambda i: (0, 0)),
            pl.BlockSpec((IN, H), lambda i: (0, 0)),
            pl.BlockSpec((1, H), lambda i: (0, 0)),
        ],
        out_specs=[qspec, qspec, qspec, qspec,
                   pl.BlockSpec((BLK, 1), lambda i: (i, 0))],
        out_shape=[qshape, qshape, qshape, qshape,
                   jax.ShapeDtypeStruct((N, 1), jnp.float32)],
    )(agg1, x, w1lt, w1rt, b1l)


def _tc2_body(a01_ref, a23_ref, q0_ref, q1_ref, q2_ref, q3_ref, ci_ref,
              bt_ref, w2lt, w2rt, b2l,
              wst, wdt, wa1t, ba1, wa2t, ba2,
              p_ref, q_ref, gsum_ref, gcnt_ref, act_ref):
    i = pl.program_id(0)
    ci = ci_ref[...]
    acc = b2l[...]
    aggs = [a01_ref[0], a01_ref[1], a23_ref[0], a23_ref[1]]
    roots = [q0_ref[...], q1_ref[...], q2_ref[...], q3_ref[...]]
    for qi in range(4):
        acc = acc + (aggs[qi] * ci) @ w2lt[qi]
        acc = acc + roots[qi] @ w2rt[qi]
    h2 = jax.nn.relu(acc)
    p_ref[...] = h2 @ wst[...]
    q_ref[...] = h2 @ wdt[...]
    bt = bt_ref[0, 0, :]
    ot = (lax.broadcasted_iota(jnp.int32, (B, BLK), 0)
          == bt[None, :]).astype(jnp.float32)

    @pl.when(i == 0)
    def _():
        gsum_ref[...] = jnp.zeros_like(gsum_ref)
        gcnt_ref[...] = jnp.zeros_like(gcnt_ref)

    gsum_ref[...] += ot @ h2
    gcnt_ref[...] += jnp.sum(ot, axis=1, keepdims=True)

    @pl.when(i == _NB - 1)
    def _():
        gm = gsum_ref[...] / jnp.maximum(gcnt_ref[...], 1.0)
        act_ref[...] = (jax.nn.relu(gm @ wa1t[...] + ba1[...])
                        @ wa2t[...] + ba2[...])


def _tc2(agg01, agg23, q0, q1, q2, q3, cinv, batch3, w2lt, w2rt, b2l,
         wst, wdt, wa1t, ba1, wa2t, ba2):
    const = lambda i: (0, 0)
    const3 = lambda i: (0, 0, 0)
    qspec = pl.BlockSpec((BLK, 16), lambda i: (i, 0))
    return pl.pallas_call(
        _tc2_body,
        grid=(_NB,),
        in_specs=[
            pl.BlockSpec((2, BLK, 16), lambda i: (0, i, 0)),
            pl.BlockSpec((2, BLK, 16), lambda i: (0, i, 0)),
            qspec, qspec, qspec, qspec,
            pl.BlockSpec((BLK, 1), lambda i: (i, 0)),
            pl.BlockSpec((1, 1, BLK), lambda i: (i, 0, 0)),
            pl.BlockSpec((4, 16, H), const3),
            pl.BlockSpec((4, 16, H), const3),
            pl.BlockSpec((1, H), const),
            pl.BlockSpec((H, 32), const),
            pl.BlockSpec((H, 32), const),
            pl.BlockSpec((H, 32), const),
            pl.BlockSpec((1, 32), const),
            pl.BlockSpec((32, 3), const),
            pl.BlockSpec((1, 3), const),
        ],
        out_specs=[
            pl.BlockSpec((BLK, 32), lambda i: (i, 0)),
            pl.BlockSpec((BLK, 32), lambda i: (i, 0)),
            pl.BlockSpec((B, H), const),
            pl.BlockSpec((B, 1), const),
            pl.BlockSpec((B, 3), const),
        ],
        out_shape=[
            jax.ShapeDtypeStruct((N, 32), jnp.float32),
            jax.ShapeDtypeStruct((N, 32), jnp.float32),
            jax.ShapeDtypeStruct((B, H), jnp.float32),
            jax.ShapeDtypeStruct((B, 1), jnp.float32),
            jax.ShapeDtypeStruct((B, 3), jnp.float32),
        ],
    )(agg01, agg23, q0, q1, q2, q3, cinv, batch3, w2lt, w2rt, b2l,
      wst, wdt, wa1t, ba1, wa2t, ba2)


EBLK = 4096
_NEB = EP // EBLK


def _tc3_body(r_ref, be1_ref, we2t_ref, be2_ref, out_ref):
    r = jax.nn.relu(r_ref[0] + r_ref[1] + be1_ref[...])
    out_ref[...] = r @ we2t_ref[...] + be2_ref[...]


def _tc3(r2, be1, we2t, be2):
    return pl.pallas_call(
        _tc3_body,
        grid=(_NEB,),
        in_specs=[
            pl.BlockSpec((2, EBLK, 32), lambda i: (0, i, 0)),
            pl.BlockSpec((1, 32), lambda i: (0, 0)),
            pl.BlockSpec((32, 1), lambda i: (0, 0)),
            pl.BlockSpec((1, 1), lambda i: (0, 0)),
        ],
        out_specs=pl.BlockSpec((EBLK, 1), lambda i: (i, 0)),
        out_shape=jax.ShapeDtypeStruct((EP, 1), jnp.float32),
    )(r2, be1, we2t, be2)


def kernel(x, edge_index, batch, W1l, b1l, W1r, W2l, b2l, W2r,
           Wa1, ba1, Wa2, ba2, We1, be1, We2, be2):
    f32 = jnp.float32
    x_pad = jnp.zeros((N, D16), f32)
    x_pad = x_pad.at[:, :IN].set(x).at[:, D16 - 1].set(1.0)

    src = jnp.concatenate(
        [edge_index[0], jnp.zeros((EP - E,), jnp.int32)]).reshape(EROWS, 128)
    dst = jnp.concatenate(
        [edge_index[1], jnp.full((EP - E,), N, jnp.int32)]).reshape(EROWS, 128)
    zeros16 = jnp.zeros((NACC, D16), f32)

    agg1 = _seg_split(x_pad, x_pad, src, dst, zeros16).reshape(2, NACC, D16)
    q0, q1, q2, q3, cinv = _tc1(agg1, x, W1l.T, W1r.T, b1l[None, :])

    agg01 = _seg_full(q0, q1, src, dst, zeros16).reshape(2, NACC, 16)
    agg23 = _seg_full(q2, q3, src, dst, zeros16).reshape(2, NACC, 16)

    w2lt = jnp.stack([W2l[:, 0:16].T, W2l[:, 16:32].T,
                      W2l[:, 32:48].T, W2l[:, 48:64].T])
    w2rt = jnp.stack([W2r[:, 0:16].T, W2r[:, 16:32].T,
                      W2r[:, 32:48].T, W2r[:, 48:64].T])
    batch3 = batch.reshape(_NB, 1, BLK)
    p_tab, q_tab, _gs, _gc, act = _tc2(
        agg01, agg23, q0, q1, q2, q3, cinv, batch3, w2lt, w2rt,
        b2l[None, :],
        We1[:, :H].T, We1[:, H:].T, Wa1.T, ba1[None, :], Wa2.T, ba2[None, :])

    r2 = _edge_gather(p_tab, q_tab, src, dst).reshape(2, EP, 32)
    el = _tc3(r2, be1[None, :], We2.T, be2[None, :])
    return act, el[:E, 0]


# speedup vs baseline: 4.9791x; 1.0935x over previous
import functools

import jax
import jax.numpy as jnp
from jax import lax
from jax.experimental import pallas as pl
from jax.experimental.pallas import tpu as pltpu
from jax.experimental.pallas import tpu_sc as plsc

N = 50000
E = 800000
IN = 11
H = 64
B = 64

D16 = 16
NACC = 50176
STRIPE = NACC // 16
EP = 819200
EROWS = EP // 128
G = 8
CHUNK = G * 128

_mesh = plsc.VectorSubcoreMesh(core_axis_name="c", subcore_axis_name="s")
_sc_params = pltpu.CompilerParams(use_tc_tiling_on_sc=False)


def _seg_chunk_loop(table, src2d, dst2d, acc, slots, row_base, niter):
    def fire(slot, k, guard_drain):
        sidx, didx, rows, gsem, ssem = slot
        if guard_drain:
            @pl.when(k >= 2)
            def _():
                for j in range(G):
                    pltpu.make_async_copy(rows.at[pl.ds(j * 128, 128)],
                                          acc.at[didx.at[j]], ssem).wait()
        rb = row_base + k * G
        pltpu.sync_copy(src2d.at[pl.ds(rb, G)], sidx)
        pltpu.sync_copy(dst2d.at[pl.ds(rb, G)], didx)
        for j in range(G):
            pltpu.async_copy(table.at[sidx.at[j]],
                             rows.at[pl.ds(j * 128, 128)], gsem)

    def work(slot):
        sidx, didx, rows, gsem, ssem = slot
        for j in range(G):
            pltpu.make_async_copy(table.at[sidx.at[j]],
                                  rows.at[pl.ds(j * 128, 128)], gsem).wait()
        for j in range(G):
            pltpu.async_copy(rows.at[pl.ds(j * 128, 128)],
                             acc.at[didx.at[j]], ssem, add=True)

    fire(slots[0], row_base * 0, False)

    def body(i, _):
        even = i % 2 == 0

        @pl.when(jnp.logical_and(even, i + 1 < niter))
        def _():
            fire(slots[1], i + 1, True)

        @pl.when(jnp.logical_and(jnp.logical_not(even), i + 1 < niter))
        def _():
            fire(slots[0], i + 1, True)

        @pl.when(even)
        def _():
            work(slots[0])

        @pl.when(jnp.logical_not(even))
        def _():
            work(slots[1])

        return 0
    lax.fori_loop(0, niter, body, 0)
    for slot in slots:
        sidx, didx, rows, gsem, ssem = slot
        for j in range(G):
            pltpu.make_async_copy(rows.at[pl.ds(j * 128, 128)],
                                  acc.at[didx.at[j]], ssem).wait()


def _make_seg_kernel(split_edges):
    rows_sc = EROWS // 2 if split_edges else EROWS
    rows_tile = rows_sc // 16
    niter = rows_tile // G

    @functools.partial(
        pl.kernel, mesh=_mesh, compiler_params=_sc_params,
        out_type=jax.ShapeDtypeStruct((2 * NACC, D16), jnp.float32),
        scratch_types=[
            pltpu.VMEM((G, 128), jnp.int32),
            pltpu.VMEM((G, 128), jnp.int32),
            pltpu.VMEM((CHUNK, D16), jnp.float32),
            pltpu.VMEM((G, 128), jnp.int32),
            pltpu.VMEM((G, 128), jnp.int32),
            pltpu.VMEM((CHUNK, D16), jnp.float32),
            pltpu.VMEM_SHARED((NACC, D16), jnp.float32),
            pltpu.SemaphoreType.DMA,
            pltpu.SemaphoreType.DMA,
            pltpu.SemaphoreType.DMA,
            pltpu.SemaphoreType.DMA,
        ],
    )
    def k(t0, t1, src2d, dst2d, zeros_hbm, out,
          si0, di0, rw0, si1, di1, rw1, acc, g0, x0, g1, x1):
        c = lax.axis_index("c")
        s = lax.axis_index("s")
        sb = s * STRIPE
        pltpu.sync_copy(zeros_hbm.at[pl.ds(sb, STRIPE)],
                        acc.at[pl.ds(sb, STRIPE)])
        plsc.subcore_barrier()
        base = s * rows_tile
        if split_edges:
            base = base + c * rows_sc
        slots = [(si0, di0, rw0, g0, x0), (si1, di1, rw1, g1, x1)]

        @pl.when(c == 0)
        def _():
            _seg_chunk_loop(t0, src2d, dst2d, acc, slots, base, niter)

        @pl.when(c == 1)
        def _():
            _seg_chunk_loop(t1, src2d, dst2d, acc, slots, base, niter)

        plsc.subcore_barrier()
        pltpu.sync_copy(acc.at[pl.ds(sb, STRIPE)],
                        out.at[pl.ds(c * NACC + sb, STRIPE)])

    return k


_seg_split = _make_seg_kernel(True)
_seg_full = _make_seg_kernel(False)

GE = 4
CHUNKE = GE * 128
_EMROWS = EROWS // GE
_EMR_TILE = _EMROWS // 32
_ESCR = ([pltpu.VMEM((1, GE, 128), jnp.int32),
          pltpu.VMEM((CHUNKE, 32), jnp.float32)] * 4
         + [pltpu.SemaphoreType.DMA] * 8)


@functools.partial(
    pl.kernel, mesh=_mesh, compiler_params=_sc_params,
    out_type=jax.ShapeDtypeStruct((2 * EP, 32), jnp.float32),
    scratch_types=_ESCR,
)
def _edge_gather(p_tab, q_tab, src3, dst3, out,
                 ip0, bp0, ip1, bp1, iq0, bq0, iq1, bq1,
                 gp0, wp0, gp1, wp1, gq0, wq0, gq1, wq1):
    c = lax.axis_index("c")
    s = lax.axis_index("s")
    w = c * 16 + s
    streams = [
        (p_tab, src3, 0, [(ip0, bp0, gp0, wp0), (ip1, bp1, gp1, wp1)]),
        (q_tab, dst3, EP, [(iq0, bq0, gq0, wq0), (iq1, bq1, gq1, wq1)]),
    ]

    def fire(stream, slot_i, k, guard_drain):
        tab, idx3, sec, slots = stream
        idx, buf, gsem, wsem = slots[slot_i]
        if guard_drain:
            @pl.when(k >= 2)
            def _():
                pltpu.make_async_copy(buf, out.at[pl.ds(sec, CHUNKE)],
                                      wsem).wait()
        pltpu.sync_copy(idx3.at[pl.ds(w * _EMR_TILE + k, 1)], idx)
        for j in range(GE):
            pltpu.async_copy(tab.at[idx.at[0, j]],
                             buf.at[pl.ds(j * 128, 128)], gsem)

    def work(stream, slot_i, k):
        tab, idx3, sec, slots = stream
        idx, buf, gsem, wsem = slots[slot_i]
        for j in range(GE):
            pltpu.make_async_copy(tab.at[idx.at[0, j]],
                                  buf.at[pl.ds(j * 128, 128)], gsem).wait()
        eb = sec + (w * _EMR_TILE + k) * CHUNKE
        pltpu.async_copy(buf, out.at[pl.ds(eb, CHUNKE)], wsem)

    for st in streams:
        fire(st, 0, 0, False)

    def body(i, _):
        even = i % 2 == 0

        @pl.when(jnp.logical_and(even, i + 1 < _EMR_TILE))
        def _():
            for st in streams:
                fire(st, 1, i + 1, True)

        @pl.when(jnp.logical_and(jnp.logical_not(even), i + 1 < _EMR_TILE))
        def _():
            for st in streams:
                fire(st, 0, i + 1, True)

        @pl.when(even)
        def _():
            for st in streams:
                work(st, 0, i)

        @pl.when(jnp.logical_not(even))
        def _():
            for st in streams:
                work(st, 1, i)

        return 0
    lax.fori_loop(0, _EMR_TILE, body, 0)
    for tab, idx3, sec, slots in streams:
        for idx, buf, gsem, wsem in slots:
            pltpu.make_async_copy(buf, out.at[pl.ds(sec, CHUNKE)],
                                  wsem).wait()


BLK = 2000
_NB = N // BLK


def _tc1_body(a_ref, x_ref, wl_ref, wr_ref, bl_ref,
              q0_ref, q1_ref, q2_ref, q3_ref, ci_ref):
    p = a_ref[0] + a_ref[1]
    cnt = p[:, D16 - 1:D16]
    ci = 1.0 / jnp.maximum(cnt, 1.0)
    mean = p[:, :IN] * ci
    h = jax.nn.relu(mean @ wl_ref[...] + x_ref[...] @ wr_ref[...] + bl_ref[...])
    q0_ref[...] = h[:, 0:16]
    q1_ref[...] = h[:, 16:32]
    q2_ref[...] = h[:, 32:48]
    q3_ref[...] = h[:, 48:64]
    ci_ref[...] = ci


def _tc1(agg1, x, w1lt, w1rt, b1l):
    qshape = jax.ShapeDtypeStruct((N, 16), jnp.float32)
    qspec = pl.BlockSpec((BLK, 16), lambda i: (i, 0))
    return pl.pallas_call(
        _tc1_body,
        grid=(_NB,),
        in_specs=[
            pl.BlockSpec((2, BLK, D16), lambda i: (0, i, 0)),
            pl.BlockSpec((BLK, IN), lambda i: (i, 0)),
            pl.BlockSpec((IN, H), lambda i: (0, 0)),
            pl.BlockSpec((IN, H), lambda i: (0, 0)),
            pl.BlockSpec((1, H), lambda i: (0, 0)),
        ],
        out_specs=[qspec, qspec, qspec, qspec,
                   pl.BlockSpec((BLK, 1), lambda i: (i, 0))],
        out_shape=[qshape, qshape, qshape, qshape,
                   jax.ShapeDtypeStruct((N, 1), jnp.float32)],
    )(agg1, x, w1lt, w1rt, b1l)


def _tc2_body(a01_ref, a23_ref, q0_ref, q1_ref, q2_ref, q3_ref, ci_ref,
              bt_ref, w2lt, w2rt, b2l,
              wst, wdt, wa1t, ba1, wa2t, ba2,
              p_ref, q_ref, gsum_ref, gcnt_ref, act_ref):
    i = pl.program_id(0)
    ci = ci_ref[...]
    acc = b2l[...]
    aggs = [a01_ref[0], a01_ref[1], a23_ref[0], a23_ref[1]]
    roots = [q0_ref[...], q1_ref[...], q2_ref[...], q3_ref[...]]
    for qi in range(4):
        acc = acc + (aggs[qi] * ci) @ w2lt[qi]
        acc = acc + roots[qi] @ w2rt[qi]
    h2 = jax.nn.relu(acc)
    p_ref[...] = h2 @ wst[...]
    q_ref[...] = h2 @ wdt[...]
    bt = bt_ref[0, 0, :]
    ot = (lax.broadcasted_iota(jnp.int32, (B, BLK), 0)
          == bt[None, :]).astype(jnp.float32)

    @pl.when(i == 0)
    def _():
        gsum_ref[...] = jnp.zeros_like(gsum_ref)
        gcnt_ref[...] = jnp.zeros_like(gcnt_ref)

    gsum_ref[...] += ot @ h2
    gcnt_ref[...] += jnp.sum(ot, axis=1, keepdims=True)

    @pl.when(i == _NB - 1)
    def _():
        gm = gsum_ref[...] / jnp.maximum(gcnt_ref[...], 1.0)
        act_ref[...] = (jax.nn.relu(gm @ wa1t[...] + ba1[...])
                        @ wa2t[...] + ba2[...])


def _tc2(agg01, agg23, q0, q1, q2, q3, cinv, batch3, w2lt, w2rt, b2l,
         wst, wdt, wa1t, ba1, wa2t, ba2):
    const = lambda i: (0, 0)
    const3 = lambda i: (0, 0, 0)
    qspec = pl.BlockSpec((BLK, 16), lambda i: (i, 0))
    return pl.pallas_call(
        _tc2_body,
        grid=(_NB,),
        in_specs=[
            pl.BlockSpec((2, BLK, 16), lambda i: (0, i, 0)),
            pl.BlockSpec((2, BLK, 16), lambda i: (0, i, 0)),
            qspec, qspec, qspec, qspec,
            pl.BlockSpec((BLK, 1), lambda i: (i, 0)),
            pl.BlockSpec((1, 1, BLK), lambda i: (i, 0, 0)),
            pl.BlockSpec((4, 16, H), const3),
            pl.BlockSpec((4, 16, H), const3),
            pl.BlockSpec((1, H), const),
            pl.BlockSpec((H, 32), const),
            pl.BlockSpec((H, 32), const),
            pl.BlockSpec((H, 32), const),
            pl.BlockSpec((1, 32), const),
            pl.BlockSpec((32, 3), const),
            pl.BlockSpec((1, 3), const),
        ],
        out_specs=[
            pl.BlockSpec((BLK, 32), lambda i: (i, 0)),
            pl.BlockSpec((BLK, 32), lambda i: (i, 0)),
            pl.BlockSpec((B, H), const),
            pl.BlockSpec((B, 1), const),
            pl.BlockSpec((B, 3), const),
        ],
        out_shape=[
            jax.ShapeDtypeStruct((N, 32), jnp.float32),
            jax.ShapeDtypeStruct((N, 32), jnp.float32),
            jax.ShapeDtypeStruct((B, H), jnp.float32),
            jax.ShapeDtypeStruct((B, 1), jnp.float32),
            jax.ShapeDtypeStruct((B, 3), jnp.float32),
        ],
    )(agg01, agg23, q0, q1, q2, q3, cinv, batch3, w2lt, w2rt, b2l,
      wst, wdt, wa1t, ba1, wa2t, ba2)


EBLK = 4096
_NEB = EP // EBLK


def _tc3_body(r_ref, be1_ref, we2t_ref, be2_ref, out_ref):
    r = jax.nn.relu(r_ref[0] + r_ref[1] + be1_ref[...])
    out_ref[...] = r @ we2t_ref[...] + be2_ref[...]


def _tc3(r2, be1, we2t, be2):
    return pl.pallas_call(
        _tc3_body,
        grid=(_NEB,),
        in_specs=[
            pl.BlockSpec((2, EBLK, 32), lambda i: (0, i, 0)),
            pl.BlockSpec((1, 32), lambda i: (0, 0)),
            pl.BlockSpec((32, 1), lambda i: (0, 0)),
            pl.BlockSpec((1, 1), lambda i: (0, 0)),
        ],
        out_specs=pl.BlockSpec((EBLK, 1), lambda i: (i, 0)),
        out_shape=jax.ShapeDtypeStruct((EP, 1), jnp.float32),
    )(r2, be1, we2t, be2)


def kernel(x, edge_index, batch, W1l, b1l, W1r, W2l, b2l, W2r,
           Wa1, ba1, Wa2, ba2, We1, be1, We2, be2):
    f32 = jnp.float32
    x_pad = jnp.zeros((N, D16), f32)
    x_pad = x_pad.at[:, :IN].set(x).at[:, D16 - 1].set(1.0)

    src = jnp.concatenate(
        [edge_index[0], jnp.zeros((EP - E,), jnp.int32)]).reshape(EROWS, 128)
    dst = jnp.concatenate(
        [edge_index[1], jnp.full((EP - E,), N, jnp.int32)]).reshape(EROWS, 128)
    zeros16 = jnp.zeros((NACC, D16), f32)

    agg1 = _seg_split(x_pad, x_pad, src, dst, zeros16).reshape(2, NACC, D16)
    q0, q1, q2, q3, cinv = _tc1(agg1, x, W1l.T, W1r.T, b1l[None, :])

    agg01 = _seg_full(q0, q1, src, dst, zeros16).reshape(2, NACC, 16)
    agg23 = _seg_full(q2, q3, src, dst, zeros16).reshape(2, NACC, 16)

    w2lt = jnp.stack([W2l[:, 0:16].T, W2l[:, 16:32].T,
                      W2l[:, 32:48].T, W2l[:, 48:64].T])
    w2rt = jnp.stack([W2r[:, 0:16].T, W2r[:, 16:32].T,
                      W2r[:, 32:48].T, W2r[:, 48:64].T])
    batch3 = batch.reshape(_NB, 1, BLK)
    p_tab, q_tab, _gs, _gc, act = _tc2(
        agg01, agg23, q0, q1, q2, q3, cinv, batch3, w2lt, w2rt,
        b2l[None, :],
        We1[:, :H].T, We1[:, H:].T, Wa1.T, ba1[None, :], Wa2.T, ba2[None, :])

    r2 = _edge_gather(p_tab, q_tab,
                      src.reshape(_EMROWS, GE, 128),
                      dst.reshape(_EMROWS, GE, 128)).reshape(2, EP, 32)
    el = _tc3(r2, be1[None, :], We2.T, be2[None, :])
    return act, el[:E, 0]


# speedup vs baseline: 4.9884x; 1.0019x over previous
import functools

import jax
import jax.numpy as jnp
from jax import lax
from jax.experimental import pallas as pl
from jax.experimental.pallas import tpu as pltpu
from jax.experimental.pallas import tpu_sc as plsc

N = 50000
E = 800000
IN = 11
H = 64
B = 64

D16 = 16
NACC = 50176
STRIPE = NACC // 16
EP = 819200
EROWS = EP // 128
G = 8
CHUNK = G * 128

_mesh = plsc.VectorSubcoreMesh(core_axis_name="c", subcore_axis_name="s")
_sc_params = pltpu.CompilerParams(use_tc_tiling_on_sc=False)


def _seg_chunk_loop(table, src2d, dst2d, acc, slots, row_base, niter):
    def fire(slot, k, guard_drain):
        sidx, didx, rows, gsem, ssem = slot
        if guard_drain:
            @pl.when(k >= 2)
            def _():
                for j in range(G):
                    pltpu.make_async_copy(rows.at[pl.ds(j * 128, 128)],
                                          acc.at[didx.at[j]], ssem).wait()
        rb = row_base + k * G
        pltpu.sync_copy(src2d.at[pl.ds(rb, G)], sidx)
        pltpu.sync_copy(dst2d.at[pl.ds(rb, G)], didx)
        for j in range(G):
            pltpu.async_copy(table.at[sidx.at[j]],
                             rows.at[pl.ds(j * 128, 128)], gsem)

    def work(slot):
        sidx, didx, rows, gsem, ssem = slot
        for j in range(G):
            pltpu.make_async_copy(table.at[sidx.at[j]],
                                  rows.at[pl.ds(j * 128, 128)], gsem).wait()
        for j in range(G):
            pltpu.async_copy(rows.at[pl.ds(j * 128, 128)],
                             acc.at[didx.at[j]], ssem, add=True)

    fire(slots[0], row_base * 0, False)

    def body(i, _):
        even = i % 2 == 0

        @pl.when(jnp.logical_and(even, i + 1 < niter))
        def _():
            fire(slots[1], i + 1, True)

        @pl.when(jnp.logical_and(jnp.logical_not(even), i + 1 < niter))
        def _():
            fire(slots[0], i + 1, True)

        @pl.when(even)
        def _():
            work(slots[0])

        @pl.when(jnp.logical_not(even))
        def _():
            work(slots[1])

        return 0
    lax.fori_loop(0, niter, body, 0)
    for slot in slots:
        sidx, didx, rows, gsem, ssem = slot
        for j in range(G):
            pltpu.make_async_copy(rows.at[pl.ds(j * 128, 128)],
                                  acc.at[didx.at[j]], ssem).wait()


def _make_seg_kernel(split_edges):
    rows_sc = EROWS // 2 if split_edges else EROWS
    rows_tile = rows_sc // 16
    niter = rows_tile // G

    @functools.partial(
        pl.kernel, mesh=_mesh, compiler_params=_sc_params,
        out_type=jax.ShapeDtypeStruct((2 * NACC, D16), jnp.float32),
        scratch_types=[
            pltpu.VMEM((G, 128), jnp.int32),
            pltpu.VMEM((G, 128), jnp.int32),
            pltpu.VMEM((CHUNK, D16), jnp.float32),
            pltpu.VMEM((G, 128), jnp.int32),
            pltpu.VMEM((G, 128), jnp.int32),
            pltpu.VMEM((CHUNK, D16), jnp.float32),
            pltpu.VMEM_SHARED((NACC, D16), jnp.float32),
            pltpu.SemaphoreType.DMA,
            pltpu.SemaphoreType.DMA,
            pltpu.SemaphoreType.DMA,
            pltpu.SemaphoreType.DMA,
        ],
    )
    def k(t0, t1, src2d, dst2d, zeros_hbm, out,
          si0, di0, rw0, si1, di1, rw1, acc, g0, x0, g1, x1):
        c = lax.axis_index("c")
        s = lax.axis_index("s")
        sb = s * STRIPE
        pltpu.sync_copy(zeros_hbm.at[pl.ds(sb, STRIPE)],
                        acc.at[pl.ds(sb, STRIPE)])
        plsc.subcore_barrier()
        base = s * rows_tile
        if split_edges:
            base = base + c * rows_sc
        slots = [(si0, di0, rw0, g0, x0), (si1, di1, rw1, g1, x1)]

        @pl.when(c == 0)
        def _():
            _seg_chunk_loop(t0, src2d, dst2d, acc, slots, base, niter)

        @pl.when(c == 1)
        def _():
            _seg_chunk_loop(t1, src2d, dst2d, acc, slots, base, niter)

        plsc.subcore_barrier()
        pltpu.sync_copy(acc.at[pl.ds(sb, STRIPE)],
                        out.at[pl.ds(c * NACC + sb, STRIPE)])

    return k


_seg_split = _make_seg_kernel(True)
_seg_full = _make_seg_kernel(False)

GE = 4
CHUNKE = GE * 128
_EMROWS = EROWS // GE
_EMR_TILE = _EMROWS // 32
_ESCR = ([pltpu.VMEM((1, GE, 128), jnp.int32),
          pltpu.VMEM((CHUNKE, 32), jnp.float32)] * 4
         + [pltpu.SemaphoreType.DMA] * 8)


@functools.partial(
    pl.kernel, mesh=_mesh, compiler_params=_sc_params,
    out_type=jax.ShapeDtypeStruct((2 * EP, 32), jnp.float32),
    scratch_types=_ESCR,
)
def _edge_gather(p0_tab, p1_tab, q0_tab, q1_tab, src3, dst3, out,
                 ip0, bp0, ip1, bp1, iq0, bq0, iq1, bq1,
                 gp0, wp0, gp1, wp1, gq0, wq0, gq1, wq1):
    c = lax.axis_index("c")
    s = lax.axis_index("s")
    w = c * 16 + s

    def make_streams(p_tab, q_tab):
        return [
            (p_tab, src3, 0, [(ip0, bp0, gp0, wp0), (ip1, bp1, gp1, wp1)]),
            (q_tab, dst3, EP, [(iq0, bq0, gq0, wq0), (iq1, bq1, gq1, wq1)]),
        ]

    def fire(stream, slot_i, k, guard_drain):
        tab, idx3, sec, slots = stream
        idx, buf, gsem, wsem = slots[slot_i]
        if guard_drain:
            @pl.when(k >= 2)
            def _():
                pltpu.make_async_copy(buf, out.at[pl.ds(sec, CHUNKE)],
                                      wsem).wait()
        pltpu.sync_copy(idx3.at[pl.ds(w * _EMR_TILE + k, 1)], idx)
        for j in range(GE):
            pltpu.async_copy(tab.at[idx.at[0, j]],
                             buf.at[pl.ds(j * 128, 128)], gsem)

    def work(stream, slot_i, k):
        tab, idx3, sec, slots = stream
        idx, buf, gsem, wsem = slots[slot_i]
        for j in range(GE):
            pltpu.make_async_copy(tab.at[idx.at[0, j]],
                                  buf.at[pl.ds(j * 128, 128)], gsem).wait()
        eb = sec + (w * _EMR_TILE + k) * CHUNKE
        pltpu.async_copy(buf, out.at[pl.ds(eb, CHUNKE)], wsem)

    def run(streams):
        for st in streams:
            fire(st, 0, 0, False)

        def body(i, _):
            even = i % 2 == 0

            @pl.when(jnp.logical_and(even, i + 1 < _EMR_TILE))
            def _():
                for st in streams:
                    fire(st, 1, i + 1, True)

            @pl.when(jnp.logical_and(jnp.logical_not(even),
                                     i + 1 < _EMR_TILE))
            def _():
                for st in streams:
                    fire(st, 0, i + 1, True)

            @pl.when(even)
            def _():
                for st in streams:
                    work(st, 0, i)

            @pl.when(jnp.logical_not(even))
            def _():
                for st in streams:
                    work(st, 1, i)

            return 0
        lax.fori_loop(0, _EMR_TILE, body, 0)
        for tab, idx3, sec, slots in streams:
            for idx, buf, gsem, wsem in slots:
                pltpu.make_async_copy(buf, out.at[pl.ds(sec, CHUNKE)],
                                      wsem).wait()

    @pl.when(c == 0)
    def _():
        run(make_streams(p0_tab, q0_tab))

    @pl.when(c == 1)
    def _():
        run(make_streams(p1_tab, q1_tab))


BLK = 2000
_NB = N // BLK


def _tc1_body(a_ref, x_ref, wl_ref, wr_ref, bl_ref,
              q0_ref, q1_ref, q2_ref, q3_ref, ci_ref):
    p = a_ref[0] + a_ref[1]
    cnt = p[:, D16 - 1:D16]
    ci = 1.0 / jnp.maximum(cnt, 1.0)
    mean = p[:, :IN] * ci
    h = jax.nn.relu(mean @ wl_ref[...] + x_ref[...] @ wr_ref[...] + bl_ref[...])
    q0_ref[...] = h[:, 0:16]
    q1_ref[...] = h[:, 16:32]
    q2_ref[...] = h[:, 32:48]
    q3_ref[...] = h[:, 48:64]
    ci_ref[...] = ci


def _tc1(agg1, x, w1lt, w1rt, b1l):
    qshape = jax.ShapeDtypeStruct((N, 16), jnp.float32)
    qspec = pl.BlockSpec((BLK, 16), lambda i: (i, 0))
    return pl.pallas_call(
        _tc1_body,
        grid=(_NB,),
        in_specs=[
            pl.BlockSpec((2, BLK, D16), lambda i: (0, i, 0)),
            pl.BlockSpec((BLK, IN), lambda i: (i, 0)),
            pl.BlockSpec((IN, H), lambda i: (0, 0)),
            pl.BlockSpec((IN, H), lambda i: (0, 0)),
            pl.BlockSpec((1, H), lambda i: (0, 0)),
        ],
        out_specs=[qspec, qspec, qspec, qspec,
                   pl.BlockSpec((BLK, 1), lambda i: (i, 0))],
        out_shape=[qshape, qshape, qshape, qshape,
                   jax.ShapeDtypeStruct((N, 1), jnp.float32)],
    )(agg1, x, w1lt, w1rt, b1l)


def _tc2_body(a01_ref, a23_ref, q0_ref, q1_ref, q2_ref, q3_ref, ci_ref,
              bt_ref, w2lt, w2rt, b2l,
              wst, wdt, wa1t, ba1, wa2t, ba2,
              p_ref, q_ref, gsum_ref, gcnt_ref, act_ref):
    i = pl.program_id(0)
    ci = ci_ref[...]
    acc = b2l[...]
    aggs = [a01_ref[0], a01_ref[1], a23_ref[0], a23_ref[1]]
    roots = [q0_ref[...], q1_ref[...], q2_ref[...], q3_ref[...]]
    for qi in range(4):
        acc = acc + (aggs[qi] * ci) @ w2lt[qi]
        acc = acc + roots[qi] @ w2rt[qi]
    h2 = jax.nn.relu(acc)
    p_ref[...] = h2 @ wst[...]
    q_ref[...] = h2 @ wdt[...]
    bt = bt_ref[0, 0, :]
    ot = (lax.broadcasted_iota(jnp.int32, (B, BLK), 0)
          == bt[None, :]).astype(jnp.float32)

    @pl.when(i == 0)
    def _():
        gsum_ref[...] = jnp.zeros_like(gsum_ref)
        gcnt_ref[...] = jnp.zeros_like(gcnt_ref)

    gsum_ref[...] += ot @ h2
    gcnt_ref[...] += jnp.sum(ot, axis=1, keepdims=True)

    @pl.when(i == _NB - 1)
    def _():
        gm = gsum_ref[...] / jnp.maximum(gcnt_ref[...], 1.0)
        act_ref[...] = (jax.nn.relu(gm @ wa1t[...] + ba1[...])
                        @ wa2t[...] + ba2[...])


def _tc2(agg01, agg23, q0, q1, q2, q3, cinv, batch3, w2lt, w2rt, b2l,
         wst, wdt, wa1t, ba1, wa2t, ba2):
    const = lambda i: (0, 0)
    const3 = lambda i: (0, 0, 0)
    qspec = pl.BlockSpec((BLK, 16), lambda i: (i, 0))
    return pl.pallas_call(
        _tc2_body,
        grid=(_NB,),
        in_specs=[
            pl.BlockSpec((2, BLK, 16), lambda i: (0, i, 0)),
            pl.BlockSpec((2, BLK, 16), lambda i: (0, i, 0)),
            qspec, qspec, qspec, qspec,
            pl.BlockSpec((BLK, 1), lambda i: (i, 0)),
            pl.BlockSpec((1, 1, BLK), lambda i: (i, 0, 0)),
            pl.BlockSpec((4, 16, H), const3),
            pl.BlockSpec((4, 16, H), const3),
            pl.BlockSpec((1, H), const),
            pl.BlockSpec((H, 32), const),
            pl.BlockSpec((H, 32), const),
            pl.BlockSpec((H, 32), const),
            pl.BlockSpec((1, 32), const),
            pl.BlockSpec((32, 3), const),
            pl.BlockSpec((1, 3), const),
        ],
        out_specs=[
            pl.BlockSpec((BLK, 32), lambda i: (i, 0)),
            pl.BlockSpec((BLK, 32), lambda i: (i, 0)),
            pl.BlockSpec((B, H), const),
            pl.BlockSpec((B, 1), const),
            pl.BlockSpec((B, 3), const),
        ],
        out_shape=[
            jax.ShapeDtypeStruct((N, 32), jnp.float32),
            jax.ShapeDtypeStruct((N, 32), jnp.float32),
            jax.ShapeDtypeStruct((B, H), jnp.float32),
            jax.ShapeDtypeStruct((B, 1), jnp.float32),
            jax.ShapeDtypeStruct((B, 3), jnp.float32),
        ],
    )(agg01, agg23, q0, q1, q2, q3, cinv, batch3, w2lt, w2rt, b2l,
      wst, wdt, wa1t, ba1, wa2t, ba2)


EBLK = 4096
_NEB = EP // EBLK


def _tc3_body(r_ref, be1_ref, we2t_ref, be2_ref, out_ref):
    r = jax.nn.relu(r_ref[0] + r_ref[1] + be1_ref[...])
    out_ref[...] = r @ we2t_ref[...] + be2_ref[...]


def _tc3(r2, be1, we2t, be2):
    return pl.pallas_call(
        _tc3_body,
        grid=(_NEB,),
        in_specs=[
            pl.BlockSpec((2, EBLK, 32), lambda i: (0, i, 0)),
            pl.BlockSpec((1, 32), lambda i: (0, 0)),
            pl.BlockSpec((32, 1), lambda i: (0, 0)),
            pl.BlockSpec((1, 1), lambda i: (0, 0)),
        ],
        out_specs=pl.BlockSpec((EBLK, 1), lambda i: (i, 0)),
        out_shape=jax.ShapeDtypeStruct((EP, 1), jnp.float32),
    )(r2, be1, we2t, be2)


def kernel(x, edge_index, batch, W1l, b1l, W1r, W2l, b2l, W2r,
           Wa1, ba1, Wa2, ba2, We1, be1, We2, be2):
    f32 = jnp.float32
    x_pad = jnp.zeros((N, D16), f32)
    x_pad = x_pad.at[:, :IN].set(x).at[:, D16 - 1].set(1.0)

    src = jnp.concatenate(
        [edge_index[0], jnp.zeros((EP - E,), jnp.int32)]).reshape(EROWS, 128)
    dst = jnp.concatenate(
        [edge_index[1], jnp.full((EP - E,), N, jnp.int32)]).reshape(EROWS, 128)
    zeros16 = jnp.zeros((NACC, D16), f32)

    x2 = jnp.stack([x_pad, x_pad])
    agg1 = _seg_split(x2[0], x2[1], src, dst, zeros16).reshape(2, NACC, D16)
    q0, q1, q2, q3, cinv = _tc1(agg1, x, W1l.T, W1r.T, b1l[None, :])

    agg01 = _seg_full(q0, q1, src, dst, zeros16).reshape(2, NACC, 16)
    agg23 = _seg_full(q2, q3, src, dst, zeros16).reshape(2, NACC, 16)

    w2lt = jnp.stack([W2l[:, 0:16].T, W2l[:, 16:32].T,
                      W2l[:, 32:48].T, W2l[:, 48:64].T])
    w2rt = jnp.stack([W2r[:, 0:16].T, W2r[:, 16:32].T,
                      W2r[:, 32:48].T, W2r[:, 48:64].T])
    batch3 = batch.reshape(_NB, 1, BLK)
    p_tab, q_tab, _gs, _gc, act = _tc2(
        agg01, agg23, q0, q1, q2, q3, cinv, batch3, w2lt, w2rt,
        b2l[None, :],
        We1[:, :H].T, We1[:, H:].T, Wa1.T, ba1[None, :], Wa2.T, ba2[None, :])

    p2 = jnp.stack([p_tab, p_tab])
    q2 = jnp.stack([q_tab, q_tab])
    r2 = _edge_gather(p2[0], p2[1], q2[0], q2[1],
                      src.reshape(_EMROWS, GE, 128),
                      dst.reshape(_EMROWS, GE, 128)).reshape(2, EP, 32)
    el = _tc3(r2, be1[None, :], We2.T, be2[None, :])
    return act, el[:E, 0]
